# baseline (speedup 1.0000x reference)
import functools

import jax
import jax.numpy as jnp
from jax import lax
from jax.experimental import pallas as pl
from jax.experimental.pallas import tpu as pltpu
from jax.experimental.pallas import tpu_sc as plsc

D = 64
LANES = 128


@functools.cache
def _info():
    info = plsc.get_sparse_core_info()
    return info, plsc.VectorSubcoreMesh(core_axis_name="c", subcore_axis_name="s")


@functools.cache
def _build_repack(vocab: int):
    info, mesh = _info()
    nw = info.num_cores * info.num_subcores
    n_full = vocab // LANES
    tail = vocab % LANES
    per_w = pl.cdiv(n_full, nw)

    @functools.partial(
        pl.kernel,
        out_type=jax.ShapeDtypeStruct((vocab * D,), jnp.float32),
        mesh=mesh,
        compiler_params=pltpu.CompilerParams(use_tc_tiling_on_sc=True, needs_layout_passes=False),
        scratch_types=[
            pltpu.VMEM((D, LANES), jnp.float32),
            pltpu.VMEM((LANES * D,), jnp.float32),
        ],
    )
    def repack(tbl_t, tail_in, packed, gbuf, tbuf):
        wid = lax.axis_index("s") * info.num_cores + lax.axis_index("c")
        iota = lax.iota(jnp.int32, 16)
        bases = [jnp.int32(64) * (jnp.int32(16 * k) + iota) for k in range(8)]

        def do_block(c):
            pltpu.sync_copy(tbl_t.at[:, pl.ds(c * LANES, LANES)], gbuf)

            def dloop(d, carry):
                for k in range(8):
                    val = gbuf[d, pl.ds(16 * k, 16)]
                    plsc.store_scatter(tbuf, [carry[k] + d], val)
                return carry
            lax.fori_loop(0, D, dloop, tuple(bases))
            pltpu.sync_copy(tbuf,
                            packed.at[pl.ds(c * (LANES * D), LANES * D)])

        def tloop(t, carry):
            c = wid * per_w + t

            @pl.when(c < n_full)
            def _():
                do_block(c)
            return carry

        lax.fori_loop(0, per_w, tloop, 0)

        if tail:
            @pl.when(wid == nw - 1)
            def _():
                nt = tail * D
                pltpu.sync_copy(tail_in, tbuf.at[pl.ds(0, nt)])
                pltpu.sync_copy(tbuf.at[pl.ds(0, nt)],
                                packed.at[pl.ds(n_full * (LANES * D), nt)])

    return repack


@functools.cache
def _build_gather(batch: int, hist: int, vocab: int):
    info, mesh = _info()
    nw = info.num_cores * info.num_subcores
    n_bblk = batch // LANES
    n_tiles = hist * n_bblk
    per_w = n_tiles // nw
    assert n_tiles % nw == 0

    @functools.partial(
        pl.kernel,
        out_type=jax.ShapeDtypeStruct((hist, D, batch), jnp.float32),
        mesh=mesh,
        compiler_params=pltpu.CompilerParams(use_tc_tiling_on_sc=True, needs_layout_passes=False),
        scratch_types=[
            pltpu.VMEM((1, LANES), jnp.int32),
            pltpu.VMEM((LANES,), jnp.int32),
            pltpu.VMEM((LANES,), jnp.int32),
            pltpu.VMEM((LANES, LANES), jnp.float32),
            pltpu.VMEM((1, D, LANES), jnp.float32),
            pltpu.SemaphoreType.DMA,
        ],
    )
    def gather(ktab, idx_t, out_t, idxb, pv, halfb, gbuf, tbuf, sem):
        wid = lax.axis_index("s") * info.num_cores + lax.axis_index("c")
        iota = lax.iota(jnp.int32, 16)
        jvecs = [jnp.int32(16 * k) + iota for k in range(8)]

        def tile_body(t, carry):
            g = wid * per_w + t
            h = g // n_bblk
            c = g % n_bblk
            pltpu.sync_copy(idx_t.at[pl.ds(h, 1), pl.ds(c * LANES, LANES)],
                            idxb)
            for k in range(8):
                v = idxb[0, pl.ds(16 * k, 16)]
                pv[pl.ds(16 * k, 16)] = lax.shift_right_logical(v, 1)
                halfb[pl.ds(16 * k, 16)] = lax.shift_left(
                    lax.bitwise_and(v, 1), 6)
            pltpu.async_copy(ktab.at[pv], gbuf, sem).wait()

            halves = tuple(halfb[pl.ds(16 * k, 16)] for k in range(8))

            def dloop(d, hv):
                for k in range(8):
                    val = plsc.load_gather(gbuf, [jvecs[k], hv[k] + d])
                    tbuf[0, d, pl.ds(16 * k, 16)] = val
                return hv
            lax.fori_loop(0, D, dloop, halves)
            pltpu.sync_copy(tbuf,
                            out_t.at[pl.ds(h, 1), :, pl.ds(c * LANES, LANES)])
            return carry

        lax.fori_loop(0, per_w, tile_body, 0)

    return gather


def kernel(input, embedding):
    batch, hist = input.shape
    vocab, d = embedding.shape
    tbl_t = jnp.swapaxes(embedding, 0, 1)
    idx_t = jnp.swapaxes(input.astype(jnp.int32), 0, 1)
    n_tail = vocab % LANES
    tail_flat = jnp.reshape(
        lax.slice(embedding, (vocab - n_tail, 0), (vocab, d)), (n_tail * d,))
    packed = _build_repack(vocab)(tbl_t, tail_flat)
    ktab = packed.reshape(vocab // 2, 2 * D)
    out_t = _build_gather(batch, hist, vocab)(ktab, idx_t)
    return jnp.transpose(out_t, (2, 0, 1))


# speedup vs baseline: 1.2916x; 1.2916x over previous
import functools

import jax
import jax.numpy as jnp
from jax import lax
from jax.experimental import pallas as pl
from jax.experimental.pallas import tpu as pltpu
from jax.experimental.pallas import tpu_sc as plsc

D = 64
LANES = 128
K1_BLKS = 3
K1_LANES = K1_BLKS * LANES
K1_WORDS = K1_LANES * D
K2_BLKS = 2
K2_LANES = K2_BLKS * LANES


@functools.cache
def _info():
    info = plsc.get_sparse_core_info()
    return info, plsc.VectorSubcoreMesh(core_axis_name="c", subcore_axis_name="s")


@functools.cache
def _build_repack(vocab: int):
    info, mesh = _info()
    nw = info.num_cores * info.num_subcores
    n_full = vocab // LANES
    tail = vocab % LANES
    n_chunks = n_full // K1_BLKS
    assert n_full % K1_BLKS == 0
    per_w = pl.cdiv(n_chunks, nw)
    per_w += per_w % 2

    @functools.partial(
        pl.kernel,
        out_type=jax.ShapeDtypeStruct((vocab * D,), jnp.float32),
        mesh=mesh,
        compiler_params=pltpu.CompilerParams(
            use_tc_tiling_on_sc=True, needs_layout_passes=False),
        scratch_types=[
            pltpu.VMEM((D, K1_LANES), jnp.float32),
            pltpu.VMEM((D, K1_LANES), jnp.float32),
            pltpu.VMEM((K1_WORDS,), jnp.float32),
            pltpu.VMEM((K1_WORDS,), jnp.float32),
            pltpu.SemaphoreType.DMA,
            pltpu.SemaphoreType.DMA,
            pltpu.SemaphoreType.DMA,
            pltpu.SemaphoreType.DMA,
        ],
    )
    def repack(tbl_t, tail_in, packed, gb0, gb1, tb0, tb1,
               gs0, gs1, os0, os1):
        wid = lax.axis_index("s") * info.num_cores + lax.axis_index("c")
        iota = lax.iota(jnp.int32, 16)
        bases = [
            jnp.int32((16 * k // LANES) * (LANES * D) + (16 * k % LANES) * D)
            + jnp.int32(64) * iota
            for k in range(K1_LANES // 16)
        ]

        def chunk_of(t):
            return wid * per_w + t

        def fire_in(t, gb, sem):
            c = chunk_of(t)
            limit = jnp.minimum(n_chunks, (wid + 1) * per_w)

            @pl.when(c < limit)
            def _():
                pltpu.async_copy(
                    tbl_t.at[:, pl.ds(c * K1_LANES, K1_LANES)], gb, sem)

        def drain_in(t, gb, sem):
            @pl.when(chunk_of(t) < n_chunks)
            def _():
                pltpu.make_async_copy(
                    tbl_t.at[:, pl.ds(0, K1_LANES)], gb, sem).wait()

        def transpose(gb, tb):
            def dloop(d, carry):
                for k in range(K1_LANES // 16):
                    val = gb[d, pl.ds(16 * k, 16)]
                    plsc.store_scatter(tb, [carry[k] + d], val)
                return carry
            lax.fori_loop(0, D, dloop, tuple(bases))

        def fire_out(t, tb, sem):
            c = chunk_of(t)

            @pl.when(c < n_chunks)
            def _():
                pltpu.async_copy(
                    tb, packed.at[pl.ds(c * K1_WORDS, K1_WORDS)], sem)

        def drain_out(t, tb, sem):
            @pl.when(chunk_of(t) < n_chunks)
            def _():
                pltpu.make_async_copy(
                    tb, packed.at[pl.ds(0, K1_WORDS)], sem).wait()

        def step(t, gb, gsem, tb, osem, first):
            drain_in(t, gb, gsem)
            if not first:
                drain_out(t - 2, tb, osem)

            @pl.when(chunk_of(t) < n_chunks)
            def _():
                transpose(gb, tb)
            fire_out(t, tb, osem)
            fire_in(t + 2, gb, gsem)

        fire_in(0, gb0, gs0)
        fire_in(1, gb1, gs1)

        def pair_body(u, carry):
            t = u * 2
            step(t, gb0, gs0, tb0, os0, first=False)
            step(t + 1, gb1, gs1, tb1, os1, first=False)
            return carry

        step(0, gb0, gs0, tb0, os0, first=True)
        step(1, gb1, gs1, tb1, os1, first=True)
        lax.fori_loop(1, per_w // 2, pair_body, 0)
        drain_out(per_w - 2, tb0, os0)
        drain_out(per_w - 1, tb1, os1)

        if tail:
            @pl.when(wid == nw - 1)
            def _():
                nt = tail * D
                pltpu.sync_copy(tail_in, tb0.at[pl.ds(0, nt)])
                pltpu.sync_copy(tb0.at[pl.ds(0, nt)],
                                packed.at[pl.ds(n_full * (LANES * D), nt)])

    return repack


@functools.cache
def _build_gather(batch: int, hist: int, vocab: int):
    info, mesh = _info()
    nw = info.num_cores * info.num_subcores
    n_sblk = batch // K2_LANES
    n_tiles = hist * n_sblk
    per_w = n_tiles // nw
    assert n_tiles % nw == 0 and per_w % 2 == 0
    nk = K2_LANES // 16

    @functools.partial(
        pl.kernel,
        out_type=jax.ShapeDtypeStruct((hist, D, batch), jnp.float32),
        mesh=mesh,
        compiler_params=pltpu.CompilerParams(
            use_tc_tiling_on_sc=True, needs_layout_passes=False),
        scratch_types=[
            pltpu.VMEM((1, K2_LANES), jnp.int32),
            pltpu.VMEM((1, K2_LANES), jnp.int32),
            pltpu.VMEM((K2_LANES,), jnp.int32),
            pltpu.VMEM((K2_LANES,), jnp.int32),
            pltpu.VMEM((K2_LANES, LANES), jnp.float32),
            pltpu.VMEM((K2_LANES, LANES), jnp.float32),
            pltpu.VMEM((1, D, K2_LANES), jnp.float32),
            pltpu.VMEM((1, D, K2_LANES), jnp.float32),
            pltpu.SemaphoreType.DMA,
            pltpu.SemaphoreType.DMA,
            pltpu.SemaphoreType.DMA,
            pltpu.SemaphoreType.DMA,
        ],
    )
    def gather(ktab, idx_t, out_t, ib0, ib1, pv0, pv1, gb0, gb1,
               tb0, tb1, gs0, gs1, os0, os1):
        wid = lax.axis_index("s") * info.num_cores + lax.axis_index("c")
        iota = lax.iota(jnp.int32, 16)
        jvecs = [jnp.int32(16 * k) + iota for k in range(nk)]

        def hc_of(t):
            g = wid * per_w + t
            return g // n_sblk, g % n_sblk

        def fire_gather(t, ib, pv, gb, sem):
            h, c = hc_of(t)
            pltpu.sync_copy(
                idx_t.at[pl.ds(h, 1), pl.ds(c * K2_LANES, K2_LANES)], ib)
            for k in range(nk):
                v = ib[0, pl.ds(16 * k, 16)]
                pv[pl.ds(16 * k, 16)] = lax.shift_right_logical(v, 1)
            for blk in range(K2_BLKS):
                pltpu.async_copy(
                    ktab.at[pv.at[pl.ds(blk * LANES, LANES)]],
                    gb.at[pl.ds(blk * LANES, LANES)], sem)

        def drain_gather(gb, sem):
            pltpu.make_async_copy(
                ktab.at[pl.ds(0, K2_LANES)], gb, sem).wait()

        def transpose(ib, gb, tb):
            halves = tuple(
                lax.shift_left(
                    lax.bitwise_and(ib[0, pl.ds(16 * k, 16)], 1), 6)
                for k in range(nk))

            def dloop(d, hv):
                for k in range(nk):
                    val = plsc.load_gather(gb, [jvecs[k], hv[k] + d])
                    tb[0, d, pl.ds(16 * k, 16)] = val
                return hv
            lax.fori_loop(0, D, dloop, halves)

        def fire_out(t, tb, sem):
            h, c = hc_of(t)
            pltpu.async_copy(
                tb, out_t.at[pl.ds(h, 1), :,
                             pl.ds(c * K2_LANES, K2_LANES)], sem)

        def drain_out(tb, sem):
            pltpu.make_async_copy(
                tb, out_t.at[pl.ds(0, 1), :, pl.ds(0, K2_LANES)], sem).wait()

        def step(t, ib, pv, gb, gsem, tb, osem, first, last):
            drain_gather(gb, gsem)
            if not first:
                drain_out(tb, osem)
            transpose(ib, gb, tb)
            fire_out(t, tb, osem)
            if not last:
                fire_gather(t + 2, ib, pv, gb, gsem)

        fire_gather(0, ib0, pv0, gb0, gs0)
        fire_gather(1, ib1, pv1, gb1, gs1)
        step(0, ib0, pv0, gb0, gs0, tb0, os0, first=True, last=False)
        step(1, ib1, pv1, gb1, gs1, tb1, os1, first=True, last=False)

        def pair_body(u, carry):
            t = u * 2
            step(t, ib0, pv0, gb0, gs0, tb0, os0, first=False, last=False)
            step(t + 1, ib1, pv1, gb1, gs1, tb1, os1, first=False, last=False)
            return carry

        lax.fori_loop(1, per_w // 2 - 1, pair_body, 0)
        t = per_w - 2
        step(t, ib0, pv0, gb0, gs0, tb0, os0, first=False, last=True)
        step(t + 1, ib1, pv1, gb1, gs1, tb1, os1, first=False, last=True)
        drain_out(tb0, os0)
        drain_out(tb1, os1)

    return gather


def kernel(input, embedding):
    batch, hist = input.shape
    vocab, d = embedding.shape
    tbl_t = jnp.swapaxes(embedding, 0, 1)
    idx_t = jnp.swapaxes(input.astype(jnp.int32), 0, 1)
    n_tail = vocab % LANES
    tail_flat = jnp.reshape(
        lax.slice(embedding, (vocab - n_tail, 0), (vocab, d)), (n_tail * d,))
    packed = _build_repack(vocab)(tbl_t, tail_flat)
    ktab = packed.reshape(vocab // 2, 2 * D)
    out_t = _build_gather(batch, hist, vocab)(ktab, idx_t)
    return jnp.transpose(out_t, (2, 0, 1))


# speedup vs baseline: 2.0415x; 1.5806x over previous
import functools

import jax
import jax.numpy as jnp
from jax import lax
from jax.experimental import pallas as pl
from jax.experimental.pallas import tpu as pltpu
from jax.experimental.pallas import tpu_sc as plsc

D = 64
LANES = 128
K1_BLKS = 3
K1_LANES = K1_BLKS * LANES
K1_WORDS = K1_LANES * D
K2_BLKS = 2
K2_LANES = K2_BLKS * LANES


@functools.cache
def _info():
    info = plsc.get_sparse_core_info()
    return info, plsc.VectorSubcoreMesh(core_axis_name="c", subcore_axis_name="s")


@functools.cache
def _build_repack(vocab: int):
    info, mesh = _info()
    nw = info.num_cores * info.num_subcores
    n_full = vocab // LANES
    tail = vocab % LANES
    n_chunks = n_full // K1_BLKS
    assert n_full % K1_BLKS == 0
    per_w = pl.cdiv(n_chunks, nw)
    per_w += per_w % 2

    @functools.partial(
        pl.kernel,
        out_type=jax.ShapeDtypeStruct((vocab * D,), jnp.float32),
        mesh=mesh,
        compiler_params=pltpu.CompilerParams(
            use_tc_tiling_on_sc=True, needs_layout_passes=False),
        scratch_types=[
            pltpu.VMEM((D, K1_LANES), jnp.float32),
            pltpu.VMEM((D, K1_LANES), jnp.float32),
            pltpu.VMEM((K1_WORDS,), jnp.float32),
            pltpu.VMEM((K1_WORDS,), jnp.float32),
            pltpu.SemaphoreType.DMA,
            pltpu.SemaphoreType.DMA,
            pltpu.SemaphoreType.DMA,
            pltpu.SemaphoreType.DMA,
        ],
    )
    def repack(tbl_t, tail_in, packed, gb0, gb1, tb0, tb1,
               gs0, gs1, os0, os1):
        wid = lax.axis_index("s") * info.num_cores + lax.axis_index("c")
        iota = lax.iota(jnp.int32, 16)
        bases = [
            jnp.int32((16 * k // LANES) * (LANES * D) + (16 * k % LANES) * D)
            + jnp.int32(64) * iota
            for k in range(K1_LANES // 16)
        ]

        def chunk_of(t):
            return wid * per_w + t

        def fire_in(t, gb, sem):
            c = chunk_of(t)
            limit = jnp.minimum(n_chunks, (wid + 1) * per_w)

            @pl.when(c < limit)
            def _():
                pltpu.async_copy(
                    tbl_t.at[:, pl.ds(c * K1_LANES, K1_LANES)], gb, sem)

        def drain_in(t, gb, sem):
            @pl.when(chunk_of(t) < n_chunks)
            def _():
                pltpu.make_async_copy(
                    tbl_t.at[:, pl.ds(0, K1_LANES)], gb, sem).wait()

        def transpose(gb, tb):
            @plsc.parallel_loop(0, D, unroll=8, carry=tuple(bases))
            def dloop(d, carry):
                for k in range(K1_LANES // 16):
                    val = gb[d, pl.ds(16 * k, 16)]
                    plsc.store_scatter(tb, [carry[k] + d], val)
                return carry

        def fire_out(t, tb, sem):
            c = chunk_of(t)

            @pl.when(c < n_chunks)
            def _():
                pltpu.async_copy(
                    tb, packed.at[pl.ds(c * K1_WORDS, K1_WORDS)], sem)

        def drain_out(t, tb, sem):
            @pl.when(chunk_of(t) < n_chunks)
            def _():
                pltpu.make_async_copy(
                    tb, packed.at[pl.ds(0, K1_WORDS)], sem).wait()

        def step(t, gb, gsem, tb, osem, first):
            drain_in(t, gb, gsem)
            if not first:
                drain_out(t - 2, tb, osem)

            @pl.when(chunk_of(t) < n_chunks)
            def _():
                transpose(gb, tb)
            fire_out(t, tb, osem)
            fire_in(t + 2, gb, gsem)

        fire_in(0, gb0, gs0)
        fire_in(1, gb1, gs1)

        def pair_body(u, carry):
            t = u * 2
            step(t, gb0, gs0, tb0, os0, first=False)
            step(t + 1, gb1, gs1, tb1, os1, first=False)
            return carry

        step(0, gb0, gs0, tb0, os0, first=True)
        step(1, gb1, gs1, tb1, os1, first=True)
        lax.fori_loop(1, per_w // 2, pair_body, 0)
        drain_out(per_w - 2, tb0, os0)
        drain_out(per_w - 1, tb1, os1)

        if tail:
            @pl.when(wid == nw - 1)
            def _():
                nt = tail * D
                pltpu.sync_copy(tail_in, tb0.at[pl.ds(0, nt)])
                pltpu.sync_copy(tb0.at[pl.ds(0, nt)],
                                packed.at[pl.ds(n_full * (LANES * D), nt)])

    return repack


@functools.cache
def _build_gather(batch: int, hist: int, vocab: int):
    info, mesh = _info()
    nw = info.num_cores * info.num_subcores
    n_sblk = batch // K2_LANES
    n_tiles = hist * n_sblk
    per_w = n_tiles // nw
    assert n_tiles % nw == 0 and per_w % 2 == 0
    nk = K2_LANES // 16

    @functools.partial(
        pl.kernel,
        out_type=jax.ShapeDtypeStruct((hist, D, batch), jnp.float32),
        mesh=mesh,
        compiler_params=pltpu.CompilerParams(
            use_tc_tiling_on_sc=True, needs_layout_passes=False),
        scratch_types=[
            pltpu.VMEM((1, K2_LANES), jnp.int32),
            pltpu.VMEM((1, K2_LANES), jnp.int32),
            pltpu.VMEM((K2_LANES,), jnp.int32),
            pltpu.VMEM((K2_LANES,), jnp.int32),
            pltpu.VMEM((K2_LANES, LANES), jnp.float32),
            pltpu.VMEM((K2_LANES, LANES), jnp.float32),
            pltpu.VMEM((1, D, K2_LANES), jnp.float32),
            pltpu.VMEM((1, D, K2_LANES), jnp.float32),
            pltpu.SemaphoreType.DMA,
            pltpu.SemaphoreType.DMA,
            pltpu.SemaphoreType.DMA,
            pltpu.SemaphoreType.DMA,
        ],
    )
    def gather(ktab, idx_t, out_t, ib0, ib1, pv0, pv1, gb0, gb1,
               tb0, tb1, gs0, gs1, os0, os1):
        wid = lax.axis_index("s") * info.num_cores + lax.axis_index("c")
        iota = lax.iota(jnp.int32, 16)
        jvecs = [jnp.int32(16 * k) + iota for k in range(nk)]

        def hc_of(t):
            g = wid * per_w + t
            return g // n_sblk, g % n_sblk

        def fire_gather(t, ib, pv, gb, sem):
            h, c = hc_of(t)
            pltpu.sync_copy(
                idx_t.at[pl.ds(h, 1), pl.ds(c * K2_LANES, K2_LANES)], ib)
            for k in range(nk):
                v = ib[0, pl.ds(16 * k, 16)]
                pv[pl.ds(16 * k, 16)] = lax.shift_right_logical(v, 1)
            for blk in range(K2_BLKS):
                pltpu.async_copy(
                    ktab.at[pv.at[pl.ds(blk * LANES, LANES)]],
                    gb.at[pl.ds(blk * LANES, LANES)], sem)

        def drain_gather(gb, sem):
            pltpu.make_async_copy(
                ktab.at[pl.ds(0, K2_LANES)], gb, sem).wait()

        def transpose(ib, gb, tb):
            halves = tuple(
                lax.shift_left(
                    lax.bitwise_and(ib[0, pl.ds(16 * k, 16)], 1), 6)
                for k in range(nk))

            @plsc.parallel_loop(0, D, unroll=8, carry=halves)
            def dloop(d, hv):
                for k in range(nk):
                    val = plsc.load_gather(gb, [jvecs[k], hv[k] + d])
                    tb[0, d, pl.ds(16 * k, 16)] = val
                return hv

        def fire_out(t, tb, sem):
            h, c = hc_of(t)
            pltpu.async_copy(
                tb, out_t.at[pl.ds(h, 1), :,
                             pl.ds(c * K2_LANES, K2_LANES)], sem)

        def drain_out(tb, sem):
            pltpu.make_async_copy(
                tb, out_t.at[pl.ds(0, 1), :, pl.ds(0, K2_LANES)], sem).wait()

        def step(t, ib, pv, gb, gsem, tb, osem, first, last):
            drain_gather(gb, gsem)
            if not first:
                drain_out(tb, osem)
            transpose(ib, gb, tb)
            fire_out(t, tb, osem)
            if not last:
                fire_gather(t + 2, ib, pv, gb, gsem)

        fire_gather(0, ib0, pv0, gb0, gs0)
        fire_gather(1, ib1, pv1, gb1, gs1)
        step(0, ib0, pv0, gb0, gs0, tb0, os0, first=True, last=False)
        step(1, ib1, pv1, gb1, gs1, tb1, os1, first=True, last=False)

        def pair_body(u, carry):
            t = u * 2
            step(t, ib0, pv0, gb0, gs0, tb0, os0, first=False, last=False)
            step(t + 1, ib1, pv1, gb1, gs1, tb1, os1, first=False, last=False)
            return carry

        lax.fori_loop(1, per_w // 2 - 1, pair_body, 0)
        t = per_w - 2
        step(t, ib0, pv0, gb0, gs0, tb0, os0, first=False, last=True)
        step(t + 1, ib1, pv1, gb1, gs1, tb1, os1, first=False, last=True)
        drain_out(tb0, os0)
        drain_out(tb1, os1)

    return gather


def kernel(input, embedding):
    batch, hist = input.shape
    vocab, d = embedding.shape
    tbl_t = jnp.swapaxes(embedding, 0, 1)
    idx_t = jnp.swapaxes(input.astype(jnp.int32), 0, 1)
    n_tail = vocab % LANES
    tail_flat = jnp.reshape(
        lax.slice(embedding, (vocab - n_tail, 0), (vocab, d)), (n_tail * d,))
    packed = _build_repack(vocab)(tbl_t, tail_flat)
    ktab = packed.reshape(vocab // 2, 2 * D)
    out_t = _build_gather(batch, hist, vocab)(ktab, idx_t)
    return jnp.transpose(out_t, (2, 0, 1))


# speedup vs baseline: 2.1705x; 1.0632x over previous
import functools

import jax
import jax.numpy as jnp
from jax import lax
from jax.experimental import pallas as pl
from jax.experimental.pallas import tpu as pltpu
from jax.experimental.pallas import tpu_sc as plsc

D = 64
LANES = 128
K1_BLKS = 3
K1_LANES = K1_BLKS * LANES
K1_WORDS = K1_LANES * D
K2_BLKS = 2
K2_LANES = K2_BLKS * LANES


@functools.cache
def _info():
    info = plsc.get_sparse_core_info()
    return info, plsc.VectorSubcoreMesh(core_axis_name="c", subcore_axis_name="s")


TC_L = 1024


def _repack_body(x_ref, o_ref):
    x = x_ref[...]
    xt = jnp.swapaxes(x, 0, 1)
    x3 = xt.reshape(TC_L // 2, 2, D)
    o_ref[:, 0:D] = x3[:, 0, :]
    o_ref[:, D:2 * D] = x3[:, 1, :]


@functools.cache
def _build_repack(vocab: int):
    n_blocks = pl.cdiv(vocab, TC_L)
    return pl.pallas_call(
        _repack_body,
        grid=(n_blocks,),
        in_specs=[pl.BlockSpec((D, TC_L), lambda i: (0, i))],
        out_specs=pl.BlockSpec((TC_L // 2, 2 * D), lambda i: (i, 0)),
        out_shape=jax.ShapeDtypeStruct((vocab // 2, 2 * D), jnp.float32),
    )


@functools.cache
def _build_gather(batch: int, hist: int, vocab: int):
    info, mesh = _info()
    nw = info.num_cores * info.num_subcores
    n_sblk = batch // K2_LANES
    n_tiles = hist * n_sblk
    per_w = n_tiles // nw
    assert n_tiles % nw == 0 and per_w % 2 == 0
    nk = K2_LANES // 16

    @functools.partial(
        pl.kernel,
        out_type=jax.ShapeDtypeStruct((hist, D, batch), jnp.float32),
        mesh=mesh,
        compiler_params=pltpu.CompilerParams(
            use_tc_tiling_on_sc=True, needs_layout_passes=False),
        scratch_types=[
            pltpu.VMEM((1, K2_LANES), jnp.int32),
            pltpu.VMEM((1, K2_LANES), jnp.int32),
            pltpu.VMEM((K2_LANES,), jnp.int32),
            pltpu.VMEM((K2_LANES,), jnp.int32),
            pltpu.VMEM((K2_LANES, LANES), jnp.float32),
            pltpu.VMEM((K2_LANES, LANES), jnp.float32),
            pltpu.VMEM((1, D, K2_LANES), jnp.float32),
            pltpu.VMEM((1, D, K2_LANES), jnp.float32),
            pltpu.SemaphoreType.DMA,
            pltpu.SemaphoreType.DMA,
            pltpu.SemaphoreType.DMA,
            pltpu.SemaphoreType.DMA,
        ],
    )
    def gather(ktab, idx_t, out_t, ib0, ib1, pv0, pv1, gb0, gb1,
               tb0, tb1, gs0, gs1, os0, os1):
        wid = lax.axis_index("s") * info.num_cores + lax.axis_index("c")
        iota = lax.iota(jnp.int32, 16)
        jvecs = [jnp.int32(16 * k) + iota for k in range(nk)]

        def hc_of(t):
            g = wid * per_w + t
            return g // n_sblk, g % n_sblk

        def fire_gather(t, ib, pv, gb, sem):
            h, c = hc_of(t)
            pltpu.sync_copy(
                idx_t.at[pl.ds(h, 1), pl.ds(c * K2_LANES, K2_LANES)], ib)
            for k in range(nk):
                v = ib[0, pl.ds(16 * k, 16)]
                pv[pl.ds(16 * k, 16)] = lax.shift_right_logical(v, 1)
            for blk in range(K2_BLKS):
                pltpu.async_copy(
                    ktab.at[pv.at[pl.ds(blk * LANES, LANES)]],
                    gb.at[pl.ds(blk * LANES, LANES)], sem)

        def drain_gather(gb, sem):
            pltpu.make_async_copy(
                ktab.at[pl.ds(0, K2_LANES)], gb, sem).wait()

        def transpose(ib, gb, tb):
            halves = tuple(
                lax.shift_left(
                    lax.bitwise_and(ib[0, pl.ds(16 * k, 16)], 1), 6)
                for k in range(nk))

            @plsc.parallel_loop(0, D, unroll=8, carry=halves)
            def dloop(d, hv):
                for k in range(nk):
                    val = plsc.load_gather(gb, [jvecs[k], hv[k] + d])
                    tb[0, d, pl.ds(16 * k, 16)] = val
                return hv

        def fire_out(t, tb, sem):
            h, c = hc_of(t)
            pltpu.async_copy(
                tb, out_t.at[pl.ds(h, 1), :,
                             pl.ds(c * K2_LANES, K2_LANES)], sem)

        def drain_out(tb, sem):
            pltpu.make_async_copy(
                tb, out_t.at[pl.ds(0, 1), :, pl.ds(0, K2_LANES)], sem).wait()

        def step(t, ib, pv, gb, gsem, tb, osem, first, last):
            drain_gather(gb, gsem)
            if not first:
                drain_out(tb, osem)
            transpose(ib, gb, tb)
            fire_out(t, tb, osem)
            if not last:
                fire_gather(t + 2, ib, pv, gb, gsem)

        fire_gather(0, ib0, pv0, gb0, gs0)
        fire_gather(1, ib1, pv1, gb1, gs1)
        step(0, ib0, pv0, gb0, gs0, tb0, os0, first=True, last=False)
        step(1, ib1, pv1, gb1, gs1, tb1, os1, first=True, last=False)

        def pair_body(u, carry):
            t = u * 2
            step(t, ib0, pv0, gb0, gs0, tb0, os0, first=False, last=False)
            step(t + 1, ib1, pv1, gb1, gs1, tb1, os1, first=False, last=False)
            return carry

        lax.fori_loop(1, per_w // 2 - 1, pair_body, 0)
        t = per_w - 2
        step(t, ib0, pv0, gb0, gs0, tb0, os0, first=False, last=True)
        step(t + 1, ib1, pv1, gb1, gs1, tb1, os1, first=False, last=True)
        drain_out(tb0, os0)
        drain_out(tb1, os1)

    return gather


def kernel(input, embedding):
    batch, hist = input.shape
    vocab, d = embedding.shape
    tbl_t = jnp.swapaxes(embedding, 0, 1)
    idx_t = jnp.swapaxes(input.astype(jnp.int32), 0, 1)
    ktab = _build_repack(vocab)(tbl_t)
    out_t = _build_gather(batch, hist, vocab)(ktab, idx_t)
    return jnp.transpose(out_t, (2, 0, 1))


# speedup vs baseline: 2.8559x; 1.3158x over previous
import functools

import jax
import jax.numpy as jnp
from jax import lax
from jax.experimental import pallas as pl
from jax.experimental.pallas import tpu as pltpu
from jax.experimental.pallas import tpu_sc as plsc

D = 64
LANES = 128
K1_BLKS = 3
K1_LANES = K1_BLKS * LANES
K1_WORDS = K1_LANES * D
K2_BLKS = 2
K2_LANES = K2_BLKS * LANES


@functools.cache
def _info():
    info = plsc.get_sparse_core_info()
    return info, plsc.VectorSubcoreMesh(core_axis_name="c", subcore_axis_name="s")


TC_L = 4096


def _repack_body(x_ref, o_ref):
    x = x_ref[...]
    xt = jnp.swapaxes(x, 0, 1)
    x3 = xt.reshape(TC_L // 2, 2, D)
    o_ref[:, 0:D] = x3[:, 0, :]
    o_ref[:, D:2 * D] = x3[:, 1, :]


@functools.cache
def _build_repack(vocab: int):
    n_blocks = pl.cdiv(vocab, TC_L)
    return pl.pallas_call(
        _repack_body,
        grid=(n_blocks,),
        in_specs=[pl.BlockSpec((D, TC_L), lambda i: (0, i))],
        out_specs=pl.BlockSpec((TC_L // 2, 2 * D), lambda i: (i, 0)),
        out_shape=jax.ShapeDtypeStruct((vocab // 2, 2 * D), jnp.float32),
    )


@functools.cache
def _build_gather(batch: int, hist: int, vocab: int):
    info, mesh = _info()
    nw = info.num_cores * info.num_subcores
    n_sblk = batch // K2_LANES
    n_tiles = hist * n_sblk
    per_w = n_tiles // nw
    assert n_tiles % nw == 0 and per_w % 2 == 0
    nk = K2_LANES // 16

    @functools.partial(
        pl.kernel,
        out_type=jax.ShapeDtypeStruct((hist, D, batch), jnp.float32),
        mesh=mesh,
        compiler_params=pltpu.CompilerParams(
            use_tc_tiling_on_sc=True, needs_layout_passes=False),
        scratch_types=[
            pltpu.VMEM((1, K2_LANES), jnp.int32),
            pltpu.VMEM((1, K2_LANES), jnp.int32),
            pltpu.VMEM((K2_LANES,), jnp.int32),
            pltpu.VMEM((K2_LANES,), jnp.int32),
            pltpu.VMEM((K2_LANES, LANES), jnp.float32),
            pltpu.VMEM((K2_LANES, LANES), jnp.float32),
            pltpu.VMEM((1, D, K2_LANES), jnp.float32),
            pltpu.VMEM((1, D, K2_LANES), jnp.float32),
            pltpu.SemaphoreType.DMA,
            pltpu.SemaphoreType.DMA,
            pltpu.SemaphoreType.DMA,
            pltpu.SemaphoreType.DMA,
        ],
    )
    def gather(ktab, idx_t, out_t, ib0, ib1, pv0, pv1, gb0, gb1,
               tb0, tb1, gs0, gs1, os0, os1):
        wid = lax.axis_index("s") * info.num_cores + lax.axis_index("c")
        iota = lax.iota(jnp.int32, 16)
        jvecs = [jnp.int32(16 * k) + iota for k in range(nk)]

        def hc_of(t):
            g = wid * per_w + t
            return g // n_sblk, g % n_sblk

        def fire_gather(t, ib, pv, gb, sem):
            h, c = hc_of(t)
            pltpu.sync_copy(
                idx_t.at[pl.ds(h, 1), pl.ds(c * K2_LANES, K2_LANES)], ib)
            for k in range(nk):
                v = ib[0, pl.ds(16 * k, 16)]
                pv[pl.ds(16 * k, 16)] = lax.shift_right_logical(v, 1)
            for blk in range(K2_BLKS):
                pltpu.async_copy(
                    ktab.at[pv.at[pl.ds(blk * LANES, LANES)]],
                    gb.at[pl.ds(blk * LANES, LANES)], sem)

        def drain_gather(gb, sem):
            pltpu.make_async_copy(
                ktab.at[pl.ds(0, K2_LANES)], gb, sem).wait()

        def transpose(ib, gb, tb):
            halves = tuple(
                lax.shift_left(
                    lax.bitwise_and(ib[0, pl.ds(16 * k, 16)], 1), 6)
                for k in range(nk))

            @plsc.parallel_loop(0, D, unroll=8, carry=halves)
            def dloop(d, hv):
                for k in range(nk):
                    val = plsc.load_gather(gb, [jvecs[k], hv[k] + d])
                    tb[0, d, pl.ds(16 * k, 16)] = val
                return hv

        def fire_out(t, tb, sem):
            h, c = hc_of(t)
            pltpu.async_copy(
                tb, out_t.at[pl.ds(h, 1), :,
                             pl.ds(c * K2_LANES, K2_LANES)], sem)

        def drain_out(tb, sem):
            pltpu.make_async_copy(
                tb, out_t.at[pl.ds(0, 1), :, pl.ds(0, K2_LANES)], sem).wait()

        def step(t, ib, pv, gb, gsem, tb, osem, first, last):
            drain_gather(gb, gsem)
            if not first:
                drain_out(tb, osem)
            transpose(ib, gb, tb)
            fire_out(t, tb, osem)
            if not last:
                fire_gather(t + 2, ib, pv, gb, gsem)

        fire_gather(0, ib0, pv0, gb0, gs0)
        fire_gather(1, ib1, pv1, gb1, gs1)
        step(0, ib0, pv0, gb0, gs0, tb0, os0, first=True, last=False)
        step(1, ib1, pv1, gb1, gs1, tb1, os1, first=True, last=False)

        def pair_body(u, carry):
            t = u * 2
            step(t, ib0, pv0, gb0, gs0, tb0, os0, first=False, last=False)
            step(t + 1, ib1, pv1, gb1, gs1, tb1, os1, first=False, last=False)
            return carry

        lax.fori_loop(1, per_w // 2 - 1, pair_body, 0)
        t = per_w - 2
        step(t, ib0, pv0, gb0, gs0, tb0, os0, first=False, last=True)
        step(t + 1, ib1, pv1, gb1, gs1, tb1, os1, first=False, last=True)
        drain_out(tb0, os0)
        drain_out(tb1, os1)

    return gather


def kernel(input, embedding):
    batch, hist = input.shape
    vocab, d = embedding.shape
    tbl_t = jnp.swapaxes(embedding, 0, 1)
    idx_t = jnp.swapaxes(input.astype(jnp.int32), 0, 1)
    ktab = _build_repack(vocab)(tbl_t)
    out_t = _build_gather(batch, hist, vocab)(ktab, idx_t)
    return jnp.transpose(out_t, (2, 0, 1))


# speedup vs baseline: 2.9408x; 1.0297x over previous
import functools

import jax
import jax.numpy as jnp
from jax import lax
from jax.experimental import pallas as pl
from jax.experimental.pallas import tpu as pltpu
from jax.experimental.pallas import tpu_sc as plsc

D = 64
LANES = 128
K1_BLKS = 3
K1_LANES = K1_BLKS * LANES
K1_WORDS = K1_LANES * D
K2_BLKS = 2
K2_LANES = K2_BLKS * LANES


@functools.cache
def _info():
    info = plsc.get_sparse_core_info()
    return info, plsc.VectorSubcoreMesh(core_axis_name="c", subcore_axis_name="s")


TC_L = 8192


def _repack_body(x_ref, o_ref):
    x = x_ref[...]
    xt = jnp.swapaxes(x, 0, 1)
    x3 = xt.reshape(TC_L // 2, 2, D)
    o_ref[:, 0:D] = x3[:, 0, :]
    o_ref[:, D:2 * D] = x3[:, 1, :]


@functools.cache
def _build_repack(vocab: int):
    n_blocks = pl.cdiv(vocab, TC_L)
    return pl.pallas_call(
        _repack_body,
        grid=(n_blocks,),
        in_specs=[pl.BlockSpec((D, TC_L), lambda i: (0, i))],
        out_specs=pl.BlockSpec((TC_L // 2, 2 * D), lambda i: (i, 0)),
        out_shape=jax.ShapeDtypeStruct((vocab // 2, 2 * D), jnp.float32),
    )


@functools.cache
def _build_gather(batch: int, hist: int, vocab: int):
    info, mesh = _info()
    nw = info.num_cores * info.num_subcores
    n_sblk = batch // K2_LANES
    n_tiles = hist * n_sblk
    per_w = n_tiles // nw
    assert n_tiles % nw == 0 and per_w % 2 == 0
    nk = K2_LANES // 16

    @functools.partial(
        pl.kernel,
        out_type=jax.ShapeDtypeStruct((hist, D, batch), jnp.float32),
        mesh=mesh,
        compiler_params=pltpu.CompilerParams(
            use_tc_tiling_on_sc=True, needs_layout_passes=False),
        scratch_types=[
            pltpu.VMEM((1, K2_LANES), jnp.int32),
            pltpu.VMEM((1, K2_LANES), jnp.int32),
            pltpu.VMEM((K2_LANES,), jnp.int32),
            pltpu.VMEM((K2_LANES,), jnp.int32),
            pltpu.VMEM((K2_LANES, LANES), jnp.float32),
            pltpu.VMEM((K2_LANES, LANES), jnp.float32),
            pltpu.VMEM((1, D, K2_LANES), jnp.float32),
            pltpu.VMEM((1, D, K2_LANES), jnp.float32),
            pltpu.SemaphoreType.DMA,
            pltpu.SemaphoreType.DMA,
            pltpu.SemaphoreType.DMA,
            pltpu.SemaphoreType.DMA,
        ],
    )
    def gather(ktab, idx_t, out_t, ib0, ib1, pv0, pv1, gb0, gb1,
               tb0, tb1, gs0, gs1, os0, os1):
        wid = lax.axis_index("s") * info.num_cores + lax.axis_index("c")
        iota = lax.iota(jnp.int32, 16)
        jvecs = [jnp.int32(16 * k) + iota for k in range(nk)]

        def hc_of(t):
            g = wid * per_w + t
            return g // n_sblk, g % n_sblk

        def fire_gather(t, ib, pv, gb, sem):
            h, c = hc_of(t)
            pltpu.sync_copy(
                idx_t.at[pl.ds(h, 1), pl.ds(c * K2_LANES, K2_LANES)], ib)
            for k in range(nk):
                v = ib[0, pl.ds(16 * k, 16)]
                pv[pl.ds(16 * k, 16)] = lax.shift_right_logical(v, 1)
            for blk in range(K2_BLKS):
                pltpu.async_copy(
                    ktab.at[pv.at[pl.ds(blk * LANES, LANES)]],
                    gb.at[pl.ds(blk * LANES, LANES)], sem)

        def drain_gather(gb, sem):
            pltpu.make_async_copy(
                ktab.at[pl.ds(0, K2_LANES)], gb, sem).wait()

        def transpose(ib, gb, tb):
            halves = tuple(
                lax.shift_left(
                    lax.bitwise_and(ib[0, pl.ds(16 * k, 16)], 1), 6)
                for k in range(nk))

            @plsc.parallel_loop(0, D, unroll=8, carry=halves)
            def dloop(d, hv):
                for k in range(nk):
                    val = plsc.load_gather(gb, [jvecs[k], hv[k] + d])
                    tb[0, d, pl.ds(16 * k, 16)] = val
                return hv

        def fire_out(t, tb, sem):
            h, c = hc_of(t)
            pltpu.async_copy(
                tb, out_t.at[pl.ds(h, 1), :,
                             pl.ds(c * K2_LANES, K2_LANES)], sem)

        def drain_out(tb, sem):
            pltpu.make_async_copy(
                tb, out_t.at[pl.ds(0, 1), :, pl.ds(0, K2_LANES)], sem).wait()

        def step(t, ib, pv, gb, gsem, tb, osem, first, last):
            drain_gather(gb, gsem)
            if not first:
                drain_out(tb, osem)
            transpose(ib, gb, tb)
            fire_out(t, tb, osem)
            if not last:
                fire_gather(t + 2, ib, pv, gb, gsem)

        fire_gather(0, ib0, pv0, gb0, gs0)
        fire_gather(1, ib1, pv1, gb1, gs1)
        step(0, ib0, pv0, gb0, gs0, tb0, os0, first=True, last=False)
        step(1, ib1, pv1, gb1, gs1, tb1, os1, first=True, last=False)

        def pair_body(u, carry):
            t = u * 2
            step(t, ib0, pv0, gb0, gs0, tb0, os0, first=False, last=False)
            step(t + 1, ib1, pv1, gb1, gs1, tb1, os1, first=False, last=False)
            return carry

        lax.fori_loop(1, per_w // 2 - 1, pair_body, 0)
        t = per_w - 2
        step(t, ib0, pv0, gb0, gs0, tb0, os0, first=False, last=True)
        step(t + 1, ib1, pv1, gb1, gs1, tb1, os1, first=False, last=True)
        drain_out(tb0, os0)
        drain_out(tb1, os1)

    return gather


def kernel(input, embedding):
    batch, hist = input.shape
    vocab, d = embedding.shape
    tbl_t = jnp.swapaxes(embedding, 0, 1)
    idx_t = jnp.swapaxes(input.astype(jnp.int32), 0, 1)
    ktab = _build_repack(vocab)(tbl_t)
    out_t = _build_gather(batch, hist, vocab)(ktab, idx_t)
    return jnp.transpose(out_t, (2, 0, 1))


# speedup vs baseline: 3.8372x; 1.3048x over previous
import functools

import jax
import jax.numpy as jnp
from jax import lax
from jax.experimental import pallas as pl
from jax.experimental.pallas import tpu as pltpu
from jax.experimental.pallas import tpu_sc as plsc

D = 64
LANES = 128
K1_BLKS = 3
K1_LANES = K1_BLKS * LANES
K1_WORDS = K1_LANES * D
K2_BLKS = 2
K2_LANES = K2_BLKS * LANES


@functools.cache
def _info():
    info = plsc.get_sparse_core_info()
    return info, plsc.VectorSubcoreMesh(core_axis_name="c", subcore_axis_name="s")


TC_L = 8192


def _repack_body(x_ref, o_ref):
    x = x_ref[...]
    xt = jnp.swapaxes(x, 0, 1)
    x3 = xt.reshape(TC_L // 2, 2, D)
    pairs = jnp.concatenate([x3[:, 0, :], x3[:, 1, :]], axis=1)
    p3 = pairs.reshape(TC_L // 32, 16, 2 * D)
    for r in range(16):
        row = p3[:, r, :]
        if r:
            row = jnp.concatenate([row[:, r:], row[:, :r]], axis=1)
        o_ref[:, r, :] = row


@functools.cache
def _build_repack(vocab: int):
    n_blocks = pl.cdiv(vocab, TC_L)
    return pl.pallas_call(
        _repack_body,
        grid=(n_blocks,),
        in_specs=[pl.BlockSpec((D, TC_L), lambda i: (0, i))],
        out_specs=pl.BlockSpec((TC_L // 32, 16, 2 * D), lambda i: (i, 0, 0)),
        out_shape=jax.ShapeDtypeStruct((vocab // 32, 16, 2 * D), jnp.float32),
    )


@functools.cache
def _build_gather(batch: int, hist: int, vocab: int):
    info, mesh = _info()
    nw = info.num_cores * info.num_subcores
    n_sblk = batch // K2_LANES
    n_tiles = hist * n_sblk
    per_w = n_tiles // nw
    assert n_tiles % nw == 0 and per_w % 2 == 0
    nk = K2_LANES // 16

    @functools.partial(
        pl.kernel,
        out_type=jax.ShapeDtypeStruct((hist, D, batch), jnp.float32),
        mesh=mesh,
        compiler_params=pltpu.CompilerParams(
            use_tc_tiling_on_sc=True, needs_layout_passes=False),
        scratch_types=[
            pltpu.VMEM((1, K2_LANES), jnp.int32),
            pltpu.VMEM((1, K2_LANES), jnp.int32),
            pltpu.VMEM((K2_LANES,), jnp.int32),
            pltpu.VMEM((K2_LANES,), jnp.int32),
            pltpu.VMEM((K2_LANES, LANES), jnp.float32),
            pltpu.VMEM((K2_LANES, LANES), jnp.float32),
            pltpu.VMEM((1, D, K2_LANES), jnp.float32),
            pltpu.VMEM((1, D, K2_LANES), jnp.float32),
            pltpu.SemaphoreType.DMA,
            pltpu.SemaphoreType.DMA,
            pltpu.SemaphoreType.DMA,
            pltpu.SemaphoreType.DMA,
        ],
    )
    def gather(ktab, idx_t, out_t, ib0, ib1, pv0, pv1, gb0, gb1,
               tb0, tb1, gs0, gs1, os0, os1):
        wid = lax.axis_index("s") * info.num_cores + lax.axis_index("c")
        iota = lax.iota(jnp.int32, 16)
        jvecs = [jnp.int32(16 * k) + iota for k in range(nk)]

        def hc_of(t):
            g = wid * per_w + t
            return g // n_sblk, g % n_sblk

        def fire_gather(t, ib, pv, gb, sem):
            h, c = hc_of(t)
            pltpu.sync_copy(
                idx_t.at[pl.ds(h, 1), pl.ds(c * K2_LANES, K2_LANES)], ib)
            for k in range(nk):
                v = ib[0, pl.ds(16 * k, 16)]
                pv[pl.ds(16 * k, 16)] = lax.shift_right_logical(v, 1)
            for blk in range(K2_BLKS):
                pltpu.async_copy(
                    ktab.at[pv.at[pl.ds(blk * LANES, LANES)]],
                    gb.at[pl.ds(blk * LANES, LANES)], sem)

        def drain_gather(gb, sem):
            pltpu.make_async_copy(
                ktab.at[pl.ds(0, K2_LANES)], gb, sem).wait()

        def transpose(ib, gb, tb):
            def hrot(k):
                v = ib[0, pl.ds(16 * k, 16)]
                half = lax.shift_left(lax.bitwise_and(v, 1), 6)
                rot = lax.bitwise_and(lax.shift_right_logical(v, 1), 15)
                return half - rot
            halves = tuple(hrot(k) for k in range(nk))

            @plsc.parallel_loop(0, D, unroll=8, carry=halves)
            def dloop(d, hv):
                for k in range(nk):
                    col = lax.bitwise_and(hv[k] + d, 127)
                    val = plsc.load_gather(gb, [jvecs[k], col])
                    tb[0, d, pl.ds(16 * k, 16)] = val
                return hv

        def fire_out(t, tb, sem):
            h, c = hc_of(t)
            pltpu.async_copy(
                tb, out_t.at[pl.ds(h, 1), :,
                             pl.ds(c * K2_LANES, K2_LANES)], sem)

        def drain_out(tb, sem):
            pltpu.make_async_copy(
                tb, out_t.at[pl.ds(0, 1), :, pl.ds(0, K2_LANES)], sem).wait()

        def step(t, ib, pv, gb, gsem, tb, osem, first, last):
            drain_gather(gb, gsem)
            if not first:
                drain_out(tb, osem)
            transpose(ib, gb, tb)
            fire_out(t, tb, osem)
            if not last:
                fire_gather(t + 2, ib, pv, gb, gsem)

        fire_gather(0, ib0, pv0, gb0, gs0)
        fire_gather(1, ib1, pv1, gb1, gs1)
        step(0, ib0, pv0, gb0, gs0, tb0, os0, first=True, last=False)
        step(1, ib1, pv1, gb1, gs1, tb1, os1, first=True, last=False)

        def pair_body(u, carry):
            t = u * 2
            step(t, ib0, pv0, gb0, gs0, tb0, os0, first=False, last=False)
            step(t + 1, ib1, pv1, gb1, gs1, tb1, os1, first=False, last=False)
            return carry

        lax.fori_loop(1, per_w // 2 - 1, pair_body, 0)
        t = per_w - 2
        step(t, ib0, pv0, gb0, gs0, tb0, os0, first=False, last=True)
        step(t + 1, ib1, pv1, gb1, gs1, tb1, os1, first=False, last=True)
        drain_out(tb0, os0)
        drain_out(tb1, os1)

    return gather


def kernel(input, embedding):
    batch, hist = input.shape
    vocab, d = embedding.shape
    tbl_t = jnp.swapaxes(embedding, 0, 1)
    idx_t = jnp.swapaxes(input.astype(jnp.int32), 0, 1)
    ktab = _build_repack(vocab)(tbl_t).reshape(vocab // 2, 2 * D)
    out_t = _build_gather(batch, hist, vocab)(ktab, idx_t)
    return jnp.transpose(out_t, (2, 0, 1))


# speedup vs baseline: 4.5301x; 1.1806x over previous
import functools

import jax
import jax.numpy as jnp
from jax import lax
from jax.experimental import pallas as pl
from jax.experimental.pallas import tpu as pltpu
from jax.experimental.pallas import tpu_sc as plsc

D = 64
LANES = 128
K1_BLKS = 3
K1_LANES = K1_BLKS * LANES
K1_WORDS = K1_LANES * D
K2_BLKS = 2
K2_LANES = K2_BLKS * LANES


@functools.cache
def _info():
    info = plsc.get_sparse_core_info()
    return info, plsc.VectorSubcoreMesh(core_axis_name="c", subcore_axis_name="s")


TC_L = 8192


def _repack_body(x_ref, o_ref):
    x = x_ref[...]
    xt = jnp.swapaxes(x, 0, 1)
    x3 = xt.reshape(TC_L // 2, 2, D)
    pairs = jnp.concatenate([x3[:, 0, :], x3[:, 1, :]], axis=1)
    o_ref[...] = pltpu.roll(pairs, 0, 1, stride=1, stride_axis=0)


@functools.cache
def _build_repack(vocab: int):
    n_blocks = pl.cdiv(vocab, TC_L)
    return pl.pallas_call(
        _repack_body,
        grid=(n_blocks,),
        in_specs=[pl.BlockSpec((D, TC_L), lambda i: (0, i))],
        out_specs=pl.BlockSpec((TC_L // 2, 2 * D), lambda i: (i, 0)),
        out_shape=jax.ShapeDtypeStruct((vocab // 2, 2 * D), jnp.float32),
    )


@functools.cache
def _build_gather(batch: int, hist: int, vocab: int):
    info, mesh = _info()
    nw = info.num_cores * info.num_subcores
    n_sblk = batch // K2_LANES
    n_tiles = hist * n_sblk
    per_w = n_tiles // nw
    assert n_tiles % nw == 0 and per_w % 2 == 0
    nk = K2_LANES // 16

    @functools.partial(
        pl.kernel,
        out_type=jax.ShapeDtypeStruct((hist, D, batch), jnp.float32),
        mesh=mesh,
        compiler_params=pltpu.CompilerParams(
            use_tc_tiling_on_sc=True, needs_layout_passes=False),
        scratch_types=[
            pltpu.VMEM((1, K2_LANES), jnp.int32),
            pltpu.VMEM((1, K2_LANES), jnp.int32),
            pltpu.VMEM((K2_LANES,), jnp.int32),
            pltpu.VMEM((K2_LANES,), jnp.int32),
            pltpu.VMEM((K2_LANES, LANES), jnp.float32),
            pltpu.VMEM((K2_LANES, LANES), jnp.float32),
            pltpu.VMEM((1, D, K2_LANES), jnp.float32),
            pltpu.VMEM((1, D, K2_LANES), jnp.float32),
            pltpu.SemaphoreType.DMA,
            pltpu.SemaphoreType.DMA,
            pltpu.SemaphoreType.DMA,
            pltpu.SemaphoreType.DMA,
        ],
    )
    def gather(ktab, idx_t, out_t, ib0, ib1, pv0, pv1, gb0, gb1,
               tb0, tb1, gs0, gs1, os0, os1):
        wid = lax.axis_index("s") * info.num_cores + lax.axis_index("c")
        iota = lax.iota(jnp.int32, 16)
        jvecs = [jnp.int32(16 * k) + iota for k in range(nk)]

        def hc_of(t):
            g = wid * per_w + t
            return g // n_sblk, g % n_sblk

        def fire_gather(t, ib, pv, gb, sem):
            h, c = hc_of(t)
            pltpu.sync_copy(
                idx_t.at[pl.ds(h, 1), pl.ds(c * K2_LANES, K2_LANES)], ib)
            for k in range(nk):
                v = ib[0, pl.ds(16 * k, 16)]
                pv[pl.ds(16 * k, 16)] = lax.shift_right_logical(v, 1)
            for blk in range(K2_BLKS):
                pltpu.async_copy(
                    ktab.at[pv.at[pl.ds(blk * LANES, LANES)]],
                    gb.at[pl.ds(blk * LANES, LANES)], sem)

        def drain_gather(gb, sem):
            pltpu.make_async_copy(
                ktab.at[pl.ds(0, K2_LANES)], gb, sem).wait()

        def transpose(ib, gb, tb):
            def hrot(k):
                v = ib[0, pl.ds(16 * k, 16)]
                half = lax.shift_left(lax.bitwise_and(v, 1), 6)
                rot = lax.bitwise_and(lax.shift_right_logical(v, 1), 127)
                return half + rot
            halves = tuple(hrot(k) for k in range(nk))

            @plsc.parallel_loop(0, D, unroll=8, carry=halves)
            def dloop(d, hv):
                for k in range(nk):
                    col = lax.bitwise_and(hv[k] + d, 127)
                    val = plsc.load_gather(gb, [jvecs[k], col])
                    tb[0, d, pl.ds(16 * k, 16)] = val
                return hv

        def fire_out(t, tb, sem):
            h, c = hc_of(t)
            pltpu.async_copy(
                tb, out_t.at[pl.ds(h, 1), :,
                             pl.ds(c * K2_LANES, K2_LANES)], sem)

        def drain_out(tb, sem):
            pltpu.make_async_copy(
                tb, out_t.at[pl.ds(0, 1), :, pl.ds(0, K2_LANES)], sem).wait()

        def step(t, ib, pv, gb, gsem, tb, osem, first, last):
            drain_gather(gb, gsem)
            if not first:
                drain_out(tb, osem)
            transpose(ib, gb, tb)
            fire_out(t, tb, osem)
            if not last:
                fire_gather(t + 2, ib, pv, gb, gsem)

        fire_gather(0, ib0, pv0, gb0, gs0)
        fire_gather(1, ib1, pv1, gb1, gs1)
        step(0, ib0, pv0, gb0, gs0, tb0, os0, first=True, last=False)
        step(1, ib1, pv1, gb1, gs1, tb1, os1, first=True, last=False)

        def pair_body(u, carry):
            t = u * 2
            step(t, ib0, pv0, gb0, gs0, tb0, os0, first=False, last=False)
            step(t + 1, ib1, pv1, gb1, gs1, tb1, os1, first=False, last=False)
            return carry

        lax.fori_loop(1, per_w // 2 - 1, pair_body, 0)
        t = per_w - 2
        step(t, ib0, pv0, gb0, gs0, tb0, os0, first=False, last=True)
        step(t + 1, ib1, pv1, gb1, gs1, tb1, os1, first=False, last=True)
        drain_out(tb0, os0)
        drain_out(tb1, os1)

    return gather


def kernel(input, embedding):
    batch, hist = input.shape
    vocab, d = embedding.shape
    tbl_t = jnp.swapaxes(embedding, 0, 1)
    idx_t = jnp.swapaxes(input.astype(jnp.int32), 0, 1)
    ktab = _build_repack(vocab)(tbl_t)
    out_t = _build_gather(batch, hist, vocab)(ktab, idx_t)
    return jnp.transpose(out_t, (2, 0, 1))


# speedup vs baseline: 4.6011x; 1.0157x over previous
import functools

import jax
import jax.numpy as jnp
from jax import lax
from jax.experimental import pallas as pl
from jax.experimental.pallas import tpu as pltpu
from jax.experimental.pallas import tpu_sc as plsc

D = 64
LANES = 128
K1_BLKS = 3
K1_LANES = K1_BLKS * LANES
K1_WORDS = K1_LANES * D
K2_BLKS = 2
K2_LANES = K2_BLKS * LANES


@functools.cache
def _info():
    info = plsc.get_sparse_core_info()
    return info, plsc.VectorSubcoreMesh(core_axis_name="c", subcore_axis_name="s")


TC_L = 16384


def _repack_body(x_ref, o_ref):
    x = x_ref[...]
    xt = jnp.swapaxes(x, 0, 1)
    x3 = xt.reshape(TC_L // 2, 2, D)
    pairs = jnp.concatenate([x3[:, 0, :], x3[:, 1, :]], axis=1)
    o_ref[...] = pltpu.roll(pairs, 0, 1, stride=1, stride_axis=0)


@functools.cache
def _build_repack(vocab: int):
    n_blocks = pl.cdiv(vocab, TC_L)
    return pl.pallas_call(
        _repack_body,
        grid=(n_blocks,),
        in_specs=[pl.BlockSpec((D, TC_L), lambda i: (0, i))],
        out_specs=pl.BlockSpec((TC_L // 2, 2 * D), lambda i: (i, 0)),
        out_shape=jax.ShapeDtypeStruct((vocab // 2, 2 * D), jnp.float32),
    )


@functools.cache
def _build_gather(batch: int, hist: int, vocab: int):
    info, mesh = _info()
    nw = info.num_cores * info.num_subcores
    n_sblk = batch // K2_LANES
    n_tiles = hist * n_sblk
    per_w = n_tiles // nw
    assert n_tiles % nw == 0 and per_w % 2 == 0
    nk = K2_LANES // 16

    @functools.partial(
        pl.kernel,
        out_type=jax.ShapeDtypeStruct((hist, D, batch), jnp.float32),
        mesh=mesh,
        compiler_params=pltpu.CompilerParams(
            use_tc_tiling_on_sc=True, needs_layout_passes=False),
        scratch_types=[
            pltpu.VMEM((1, K2_LANES), jnp.int32),
            pltpu.VMEM((1, K2_LANES), jnp.int32),
            pltpu.VMEM((K2_LANES,), jnp.int32),
            pltpu.VMEM((K2_LANES,), jnp.int32),
            pltpu.VMEM((K2_LANES, LANES), jnp.float32),
            pltpu.VMEM((K2_LANES, LANES), jnp.float32),
            pltpu.VMEM((1, D, K2_LANES), jnp.float32),
            pltpu.VMEM((1, D, K2_LANES), jnp.float32),
            pltpu.SemaphoreType.DMA,
            pltpu.SemaphoreType.DMA,
            pltpu.SemaphoreType.DMA,
            pltpu.SemaphoreType.DMA,
        ],
    )
    def gather(ktab, idx_t, out_t, ib0, ib1, pv0, pv1, gb0, gb1,
               tb0, tb1, gs0, gs1, os0, os1):
        wid = lax.axis_index("s") * info.num_cores + lax.axis_index("c")
        iota = lax.iota(jnp.int32, 16)
        jvecs = [jnp.int32(16 * k) + iota for k in range(nk)]

        def hc_of(t):
            g = wid * per_w + t
            return g // n_sblk, g % n_sblk

        def fire_gather(t, ib, pv, gb, sem):
            h, c = hc_of(t)
            pltpu.sync_copy(
                idx_t.at[pl.ds(h, 1), pl.ds(c * K2_LANES, K2_LANES)], ib)
            for k in range(nk):
                v = ib[0, pl.ds(16 * k, 16)]
                pv[pl.ds(16 * k, 16)] = lax.shift_right_logical(v, 1)
            for blk in range(K2_BLKS):
                pltpu.async_copy(
                    ktab.at[pv.at[pl.ds(blk * LANES, LANES)]],
                    gb.at[pl.ds(blk * LANES, LANES)], sem)

        def drain_gather(gb, sem):
            pltpu.make_async_copy(
                ktab.at[pl.ds(0, K2_LANES)], gb, sem).wait()

        def transpose(ib, gb, tb):
            def hrot(k):
                v = ib[0, pl.ds(16 * k, 16)]
                half = lax.shift_left(lax.bitwise_and(v, 1), 6)
                rot = lax.bitwise_and(lax.shift_right_logical(v, 1), 127)
                return half + rot
            halves = tuple(hrot(k) for k in range(nk))

            @plsc.parallel_loop(0, D, unroll=8, carry=halves)
            def dloop(d, hv):
                for k in range(nk):
                    col = lax.bitwise_and(hv[k] + d, 127)
                    val = plsc.load_gather(gb, [jvecs[k], col])
                    tb[0, d, pl.ds(16 * k, 16)] = val
                return hv

        def fire_out(t, tb, sem):
            h, c = hc_of(t)
            pltpu.async_copy(
                tb, out_t.at[pl.ds(h, 1), :,
                             pl.ds(c * K2_LANES, K2_LANES)], sem)

        def drain_out(tb, sem):
            pltpu.make_async_copy(
                tb, out_t.at[pl.ds(0, 1), :, pl.ds(0, K2_LANES)], sem).wait()

        def step(t, ib, pv, gb, gsem, tb, osem, first, last):
            drain_gather(gb, gsem)
            if not first:
                drain_out(tb, osem)
            transpose(ib, gb, tb)
            fire_out(t, tb, osem)
            if not last:
                fire_gather(t + 2, ib, pv, gb, gsem)

        fire_gather(0, ib0, pv0, gb0, gs0)
        fire_gather(1, ib1, pv1, gb1, gs1)
        step(0, ib0, pv0, gb0, gs0, tb0, os0, first=True, last=False)
        step(1, ib1, pv1, gb1, gs1, tb1, os1, first=True, last=False)

        def pair_body(u, carry):
            t = u * 2
            step(t, ib0, pv0, gb0, gs0, tb0, os0, first=False, last=False)
            step(t + 1, ib1, pv1, gb1, gs1, tb1, os1, first=False, last=False)
            return carry

        lax.fori_loop(1, per_w // 2 - 1, pair_body, 0)
        t = per_w - 2
        step(t, ib0, pv0, gb0, gs0, tb0, os0, first=False, last=True)
        step(t + 1, ib1, pv1, gb1, gs1, tb1, os1, first=False, last=True)
        drain_out(tb0, os0)
        drain_out(tb1, os1)

    return gather


def kernel(input, embedding):
    batch, hist = input.shape
    vocab, d = embedding.shape
    tbl_t = jnp.swapaxes(embedding, 0, 1)
    idx_t = jnp.swapaxes(input.astype(jnp.int32), 0, 1)
    ktab = _build_repack(vocab)(tbl_t)
    out_t = _build_gather(batch, hist, vocab)(ktab, idx_t)
    return jnp.transpose(out_t, (2, 0, 1))


# speedup vs baseline: 6.2652x; 1.3617x over previous
import functools

import jax
import jax.numpy as jnp
from jax import lax
from jax.experimental import pallas as pl
from jax.experimental.pallas import tpu as pltpu
from jax.experimental.pallas import tpu_sc as plsc

D = 64
LANES = 128
K1_BLKS = 3
K1_LANES = K1_BLKS * LANES
K1_WORDS = K1_LANES * D
K2_BLKS = 2
K2_LANES = K2_BLKS * LANES


@functools.cache
def _info():
    info = plsc.get_sparse_core_info()
    return info, plsc.VectorSubcoreMesh(core_axis_name="c", subcore_axis_name="s")


TC_L = 16384


def _repack_body(x_ref, o_ref):
    x = x_ref[...]
    xt = jnp.swapaxes(x, 0, 1)
    pairs = jnp.concatenate(
        [xt[0:TC_L // 2], xt[TC_L // 2:TC_L]], axis=1)
    o_ref[...] = pltpu.roll(pairs, 0, 1, stride=1, stride_axis=0)


@functools.cache
def _build_repack(vocab: int):
    n_blocks = pl.cdiv(vocab, TC_L)
    n_rows = (n_blocks - 1) * (TC_L // 2) + min(
        TC_L // 2, vocab - (n_blocks - 1) * TC_L)
    return pl.pallas_call(
        _repack_body,
        grid=(n_blocks,),
        in_specs=[pl.BlockSpec((D, TC_L), lambda i: (0, i))],
        out_specs=pl.BlockSpec((TC_L // 2, 2 * D), lambda i: (i, 0)),
        out_shape=jax.ShapeDtypeStruct((n_rows, 2 * D), jnp.float32),
    )


@functools.cache
def _build_gather(batch: int, hist: int, vocab: int):
    info, mesh = _info()
    nw = info.num_cores * info.num_subcores
    n_sblk = batch // K2_LANES
    n_tiles = hist * n_sblk
    per_w = n_tiles // nw
    assert n_tiles % nw == 0 and per_w % 2 == 0
    nk = K2_LANES // 16

    @functools.partial(
        pl.kernel,
        out_type=jax.ShapeDtypeStruct((hist, D, batch), jnp.float32),
        mesh=mesh,
        compiler_params=pltpu.CompilerParams(
            use_tc_tiling_on_sc=True, needs_layout_passes=False),
        scratch_types=[
            pltpu.VMEM((1, K2_LANES), jnp.int32),
            pltpu.VMEM((1, K2_LANES), jnp.int32),
            pltpu.VMEM((K2_LANES,), jnp.int32),
            pltpu.VMEM((K2_LANES,), jnp.int32),
            pltpu.VMEM((K2_LANES, LANES), jnp.float32),
            pltpu.VMEM((K2_LANES, LANES), jnp.float32),
            pltpu.VMEM((1, D, K2_LANES), jnp.float32),
            pltpu.VMEM((1, D, K2_LANES), jnp.float32),
            pltpu.SemaphoreType.DMA,
            pltpu.SemaphoreType.DMA,
            pltpu.SemaphoreType.DMA,
            pltpu.SemaphoreType.DMA,
        ],
    )
    def gather(ktab, idx_t, out_t, ib0, ib1, pv0, pv1, gb0, gb1,
               tb0, tb1, gs0, gs1, os0, os1):
        wid = lax.axis_index("s") * info.num_cores + lax.axis_index("c")
        iota = lax.iota(jnp.int32, 16)
        jvecs = [jnp.int32(16 * k) + iota for k in range(nk)]

        def hc_of(t):
            g = wid * per_w + t
            return g // n_sblk, g % n_sblk

        def fire_gather(t, ib, pv, gb, sem):
            h, c = hc_of(t)
            pltpu.sync_copy(
                idx_t.at[pl.ds(h, 1), pl.ds(c * K2_LANES, K2_LANES)], ib)
            for k in range(nk):
                v = ib[0, pl.ds(16 * k, 16)]
                pv[pl.ds(16 * k, 16)] = lax.bitwise_or(
                    lax.shift_left(lax.shift_right_logical(v, 14), 13),
                    lax.bitwise_and(v, 8191))
            for blk in range(K2_BLKS):
                pltpu.async_copy(
                    ktab.at[pv.at[pl.ds(blk * LANES, LANES)]],
                    gb.at[pl.ds(blk * LANES, LANES)], sem)

        def drain_gather(gb, sem):
            pltpu.make_async_copy(
                ktab.at[pl.ds(0, K2_LANES)], gb, sem).wait()

        def transpose(ib, gb, tb):
            def hrot(k):
                v = ib[0, pl.ds(16 * k, 16)]
                half = lax.shift_left(
                    lax.bitwise_and(lax.shift_right_logical(v, 13), 1), 6)
                p = lax.bitwise_or(
                    lax.shift_left(lax.shift_right_logical(v, 14), 13),
                    lax.bitwise_and(v, 8191))
                rot = lax.bitwise_and(p, 127)
                return half + rot
            halves = tuple(hrot(k) for k in range(nk))

            @plsc.parallel_loop(0, D, unroll=8, carry=halves)
            def dloop(d, hv):
                for k in range(nk):
                    col = lax.bitwise_and(hv[k] + d, 127)
                    val = plsc.load_gather(gb, [jvecs[k], col])
                    tb[0, d, pl.ds(16 * k, 16)] = val
                return hv

        def fire_out(t, tb, sem):
            h, c = hc_of(t)
            pltpu.async_copy(
                tb, out_t.at[pl.ds(h, 1), :,
                             pl.ds(c * K2_LANES, K2_LANES)], sem)

        def drain_out(tb, sem):
            pltpu.make_async_copy(
                tb, out_t.at[pl.ds(0, 1), :, pl.ds(0, K2_LANES)], sem).wait()

        def step(t, ib, pv, gb, gsem, tb, osem, first, last):
            drain_gather(gb, gsem)
            if not first:
                drain_out(tb, osem)
            transpose(ib, gb, tb)
            fire_out(t, tb, osem)
            if not last:
                fire_gather(t + 2, ib, pv, gb, gsem)

        fire_gather(0, ib0, pv0, gb0, gs0)
        fire_gather(1, ib1, pv1, gb1, gs1)
        step(0, ib0, pv0, gb0, gs0, tb0, os0, first=True, last=False)
        step(1, ib1, pv1, gb1, gs1, tb1, os1, first=True, last=False)

        def pair_body(u, carry):
            t = u * 2
            step(t, ib0, pv0, gb0, gs0, tb0, os0, first=False, last=False)
            step(t + 1, ib1, pv1, gb1, gs1, tb1, os1, first=False, last=False)
            return carry

        lax.fori_loop(1, per_w // 2 - 1, pair_body, 0)
        t = per_w - 2
        step(t, ib0, pv0, gb0, gs0, tb0, os0, first=False, last=True)
        step(t + 1, ib1, pv1, gb1, gs1, tb1, os1, first=False, last=True)
        drain_out(tb0, os0)
        drain_out(tb1, os1)

    return gather


def kernel(input, embedding):
    batch, hist = input.shape
    vocab, d = embedding.shape
    tbl_t = jnp.swapaxes(embedding, 0, 1)
    idx_t = jnp.swapaxes(input.astype(jnp.int32), 0, 1)
    ktab = _build_repack(vocab)(tbl_t)
    out_t = _build_gather(batch, hist, vocab)(ktab, idx_t)
    return jnp.transpose(out_t, (2, 0, 1))
